# baseline (speedup 1.0000x reference)
import functools

import jax
import jax.numpy as jnp
from jax import lax
from jax.experimental import pallas as pl
from jax.experimental.pallas import tpu as pltpu
from jax.experimental.pallas import tpu_sc as plsc

N_NODES = 10000
D = 128
NS = 16
CHUNK = 128
CPW = 160
EPW = CHUNK * CPW
E_PAD = NS * EPW
ACC_ROWS = 10240
ROWS_PER_TILE = ACC_ROWS // NS
PAD_DST = N_NODES

BM = 1000


def _seg_sum_sc(h, packed3):
    mesh = plsc.VectorSubcoreMesh(core_axis_name="c", subcore_axis_name="s",
                                  num_cores=1, num_subcores=NS)

    @functools.partial(
        pl.kernel,
        out_type=jax.ShapeDtypeStruct((ACC_ROWS, D), jnp.float32),
        mesh=mesh,
        scratch_types=[
            pltpu.VMEM((CHUNK,), jnp.int32),
            pltpu.VMEM((CHUNK,), jnp.int32),
            pltpu.VMEM((CHUNK,), jnp.int32),
            pltpu.VMEM((CHUNK,), jnp.int32),
            pltpu.VMEM((CHUNK,), jnp.int32),
            pltpu.VMEM((CHUNK,), jnp.int32),
            pltpu.VMEM((CHUNK, D), jnp.float32),
            pltpu.VMEM((CHUNK, D), jnp.float32),
            pltpu.VMEM_SHARED((ACC_ROWS, D), jnp.float32),
            pltpu.SemaphoreType.DMA,
            pltpu.SemaphoreType.DMA,
            pltpu.SemaphoreType.DMA,
            pltpu.SemaphoreType.DMA,
        ],
    )
    def k(h_hbm, packed_hbm, out_hbm, pb0, pb1, sbuf0, sbuf1, dbuf0, dbuf1,
          msg0, msg1, acc, si0, si1, sg0, sg1):
        sid = lax.axis_index("s")
        my_packed = packed_hbm.at[sid]

        pltpu.async_copy(my_packed.at[0], pb0, si0)
        pltpu.async_copy(my_packed.at[1], pb1, si1)

        zero = jnp.zeros((16,), jnp.float32)

        def zrow(i, carry):
            for j in range(D // 16):
                msg0[i, pl.ds(j * 16, 16)] = zero
            return carry

        lax.fori_loop(0, CHUNK, zrow, 0)
        base = sid * ROWS_PER_TILE
        for kk in range(ROWS_PER_TILE // CHUNK):
            pltpu.sync_copy(msg0, acc.at[pl.ds(base + kk * CHUNK, CHUNK)])
        plsc.subcore_barrier()

        def unpack(pb, sbuf, dbuf):
            for j in range(CHUNK // 16):
                v = pb[pl.ds(j * 16, 16)]
                sbuf[pl.ds(j * 16, 16)] = lax.bitwise_and(v, 0xFFFF)
                dbuf[pl.ds(j * 16, 16)] = lax.shift_right_logical(v, 16)

        def idx_wait(pb, sem):
            pltpu.make_async_copy(my_packed.at[0], pb, sem).wait()

        def gather_wait(sbuf, msg, sem):
            pltpu.make_async_copy(h_hbm.at[sbuf], msg, sem).wait()

        idx_wait(pb0, si0)
        unpack(pb0, sbuf0, dbuf0)
        pltpu.async_copy(h_hbm.at[sbuf0], msg0, sg0)
        pltpu.async_copy(my_packed.at[2], pb0, si0)
        idx_wait(pb1, si1)
        unpack(pb1, sbuf1, dbuf1)
        pltpu.async_copy(h_hbm.at[sbuf1], msg1, sg1)
        pltpu.async_copy(my_packed.at[3], pb1, si1)

        def body(i, carry):
            c = i * 2
            gather_wait(sbuf0, msg0, sg0)
            pltpu.sync_copy(msg0, acc.at[dbuf0], add=True)
            idx_wait(pb0, si0)
            unpack(pb0, sbuf0, dbuf0)
            pltpu.async_copy(h_hbm.at[sbuf0], msg0, sg0)
            pltpu.async_copy(my_packed.at[jnp.minimum(c + 4, CPW - 1)],
                             pb0, si0)
            gather_wait(sbuf1, msg1, sg1)
            pltpu.sync_copy(msg1, acc.at[dbuf1], add=True)
            idx_wait(pb1, si1)
            unpack(pb1, sbuf1, dbuf1)
            pltpu.async_copy(h_hbm.at[sbuf1], msg1, sg1)
            pltpu.async_copy(my_packed.at[jnp.minimum(c + 5, CPW - 1)],
                             pb1, si1)
            return carry

        lax.fori_loop(0, CPW // 2, body, 0)
        gather_wait(sbuf0, msg0, sg0)
        gather_wait(sbuf1, msg1, sg1)
        idx_wait(pb0, si0)
        idx_wait(pb1, si1)
        plsc.subcore_barrier()

        for kk in range(ROWS_PER_TILE // CHUNK):
            r = base + kk * CHUNK
            pltpu.sync_copy(acc.at[pl.ds(r, CHUNK)], msg0)
            pltpu.sync_copy(msg0, out_hbm.at[pl.ds(r, CHUNK)])

    return k(h, packed3)


def _mm(x, W):
    M = x.shape[0]

    def kfn(x_ref, w_ref, o_ref):
        o_ref[...] = jnp.dot(x_ref[...], w_ref[...],
                             preferred_element_type=jnp.float32)

    return pl.pallas_call(
        kfn,
        grid=(M // BM,),
        in_specs=[pl.BlockSpec((BM, D), lambda i: (i, 0)),
                  pl.BlockSpec((D, D), lambda i: (0, 0))],
        out_specs=pl.BlockSpec((BM, D), lambda i: (i, 0)),
        out_shape=jax.ShapeDtypeStruct((M, D), jnp.float32),
    )(x, W)


def _comb_mm(acc, b2d, W):
    def kfn(a_ref, b_ref, w_ref, o_ref):
        h = jnp.maximum(a_ref[...] + b_ref[...], 0.0)
        o_ref[...] = jnp.dot(h, w_ref[...],
                             preferred_element_type=jnp.float32)

    return pl.pallas_call(
        kfn,
        grid=(N_NODES // BM,),
        in_specs=[pl.BlockSpec((BM, D), lambda i: (i, 0)),
                  pl.BlockSpec((1, D), lambda i: (0, 0)),
                  pl.BlockSpec((D, D), lambda i: (0, 0))],
        out_specs=pl.BlockSpec((BM, D), lambda i: (i, 0)),
        out_shape=jax.ShapeDtypeStruct((N_NODES, D), jnp.float32),
    )(acc, b2d, W)


def _comb(acc, b2d):
    def kfn(a_ref, b_ref, o_ref):
        o_ref[...] = jnp.maximum(a_ref[...] + b_ref[...], 0.0)

    return pl.pallas_call(
        kfn,
        grid=(N_NODES // BM,),
        in_specs=[pl.BlockSpec((BM, D), lambda i: (i, 0)),
                  pl.BlockSpec((1, D), lambda i: (0, 0))],
        out_specs=pl.BlockSpec((BM, D), lambda i: (i, 0)),
        out_shape=jax.ShapeDtypeStruct((N_NODES, D), jnp.float32),
    )(acc, b2d)


def kernel(x, edge_index, W1, b1, W2, b2):
    src = edge_index[0].astype(jnp.int32)
    dst = edge_index[1].astype(jnp.int32)
    n_edges = src.shape[0]
    pad = E_PAD - n_edges
    packed = jnp.bitwise_or(jnp.left_shift(dst, 16), src)
    packed3 = jnp.concatenate(
        [packed, jnp.full((pad,), PAD_DST << 16, jnp.int32)]
    ).reshape(NS, CPW, CHUNK)
    b1r = b1.reshape(1, D)
    b2r = b2.reshape(1, D)

    h1 = _mm(x, W1)
    acc1 = _seg_sum_sc(h1, packed3)
    h2 = _comb_mm(acc1, b1r, W2)
    acc2 = _seg_sum_sc(h2, packed3)
    return _comb(acc2, b2r)
